# baseline (speedup 1.0000x reference)
import functools

import jax
import jax.numpy as jnp
from jax import lax
from jax.experimental import pallas as pl
from jax.experimental.pallas import tpu as pltpu
from jax.experimental.pallas import tpu_sc as plsc

N = 10000
E = 320000
D_IN = 128
HEADS = 8
HEAD_DIM = 32
EMB = 256
HIDDEN = 512
B = 50

NW = 32
ET = E // 128
TPW = ET // NW
EXTRA = ET - TPW * NW
EMAX = (TPW + 1) * 128
NV78 = TPW * 8
HSW = 272
HSP = 384
SB = 64
BUF = EMAX + 128


def _dense_tc(x, W, A_src, A_dst):
    BLK = 2000

    def body(x_ref, w_ref, as_ref, ad_ref, o_ref):
        W_ = w_ref[...]
        Wf = jnp.concatenate(
            [W_,
             jnp.dot(W_, as_ref[...], preferred_element_type=jnp.float32),
             jnp.dot(W_, ad_ref[...], preferred_element_type=jnp.float32),
             jnp.zeros((D_IN, HSP - HSW), jnp.float32)],
            axis=1)
        o_ref[...] = jnp.dot(x_ref[...], Wf, preferred_element_type=jnp.float32)

    return pl.pallas_call(
        body,
        grid=(N // BLK,),
        in_specs=[
            pl.BlockSpec((BLK, D_IN), lambda i: (i, 0)),
            pl.BlockSpec((D_IN, EMB), lambda i: (0, 0)),
            pl.BlockSpec((EMB, HEADS), lambda i: (0, 0)),
            pl.BlockSpec((EMB, HEADS), lambda i: (0, 0)),
        ],
        out_specs=pl.BlockSpec((BLK, HSP), lambda i: (i, 0)),
        out_shape=jax.ShapeDtypeStruct((N, HSP), jnp.float32),
    )(x, W, A_src, A_dst)


def _make_sc_kernel():
    mesh = plsc.VectorSubcoreMesh(core_axis_name="c", subcore_axis_name="s")

    @functools.partial(
        pl.kernel,
        mesh=mesh,
        out_type=jax.ShapeDtypeStruct((NW, B, HSW), jnp.float32),
        compiler_params=pltpu.CompilerParams(needs_layout_passes=False),
        scratch_types=[
            pltpu.VMEM((N,), jnp.int32),
            pltpu.VMEM((64,), jnp.int32),
            pltpu.VMEM((2, EMAX), jnp.int32),
            pltpu.VMEM((BUF,), jnp.int32),
            pltpu.VMEM((BUF,), jnp.int32),
            pltpu.VMEM((2, SB, HSP), jnp.float32),
            pltpu.VMEM((HEADS, 64), jnp.float32),
            pltpu.VMEM((16, 16), jnp.float32),
            pltpu.VMEM((B, HSW), jnp.float32),
            pltpu.SemaphoreType.DMA,
            pltpu.SemaphoreType.DMA,
            pltpu.SemaphoreType.DMA,
            pltpu.SemaphoreType.DMA,
        ],
    )
    def sc_kernel(hs_hbm, edge_hbm, adj_hbm, neg_hbm, zero_hbm, out_hbm,
                  slot_tab, adj_v, e0, src_buf, slot_buf, hbuf2,
                  sdst_buf, ex_buf, acc, sem, semE, sem_slot, sem_acc):
        cid = lax.axis_index("c")
        sid = lax.axis_index("s")
        wid = sid * 2 + cid
        bt = TPW * wid + jnp.minimum(wid, EXTRA)
        base = bt * 128
        iota = lax.iota(jnp.int32, 16)
        zf = jnp.zeros((16,), jnp.float32)

        h_slot = pltpu.async_copy(neg_hbm, slot_tab, sem_slot)
        h_acc = pltpu.async_copy(zero_hbm, acc, sem_acc)
        for r in range(8, 16):
            ex_buf[r] = zf

        @pl.when(wid < NW - 1)
        def _():
            pltpu.async_copy(edge_hbm.at[:, pl.ds(base, EMAX)], e0, semE)

        @pl.when(wid == NW - 1)
        def _():
            pltpu.async_copy(edge_hbm.at[:, pl.ds(base, TPW * 128)],
                             e0.at[:, pl.ds(0, TPW * 128)], semE)
            pltpu.async_copy(edge_hbm.at[:, pl.ds(0, 128)],
                             e0.at[:, pl.ds(TPW * 128, 128)], semE)

        scope_p0a = jax.named_scope("p0a_adj")
        scope_p0a.__enter__()
        pltpu.sync_copy(adj_hbm, adj_v)
        h_tgt = pltpu.async_copy(hs_hbm.at[adj_v], hbuf2.at[0], sem)
        h_slot.wait()
        for t in range(4):
            av = adj_v[pl.ds(t * 16, 16)]
            sl = iota + (t * 16)
            plsc.store_scatter(slot_tab, [av], sl, mask=sl < B)
        scope_p0a.__exit__(None, None, None)
        scope_p0b = jax.named_scope("p0b_sdst")
        scope_p0b.__enter__()
        h_tgt.wait()
        for hh in range(HEADS):
            col = jnp.full((16,), EMB + HEADS + hh, jnp.int32)
            for t in range(4):
                v = plsc.load_gather(hbuf2.at[0], [iota + t * 16, col])
                sdst_buf[hh, pl.ds(t * 16, 16)] = v

        scope_p0b.__exit__(None, None, None)
        with jax.named_scope("p0c_ewait"):
            pltpu.make_async_copy(edge_hbm.at[:, pl.ds(0, EMAX)],
                                  e0, semE).wait()

        def vec_body(vi, kk):
            srcv = e0[0, pl.ds(vi * 16, 16)]
            dstv = e0[1, pl.ds(vi * 16, 16)]
            slv = plsc.load_gather(slot_tab, [dstv])
            m = slv >= 0
            mi = m.astype(jnp.int32)
            incl = jnp.cumsum(mi)
            idxv = kk + (incl - mi)
            plsc.store_scatter(src_buf, [idxv], srcv, mask=m)
            plsc.store_scatter(slot_buf, [idxv], slv, mask=m)
            return kk + incl[15]

        scope_p1 = jax.named_scope("p1_filter")
        scope_p1.__enter__()
        k = lax.fori_loop(0, NV78, vec_body, jnp.int32(0), unroll=8)
        nv = NV78 + 8 * jnp.int32(wid < EXTRA)
        k = lax.fori_loop(NV78, nv, vec_body, k)
        scope_p1.__exit__(None, None, None)
        h_acc.wait()

        zi = jnp.zeros((16,), jnp.int32)
        for t in range(SB // 16):
            src_buf[pl.ds(k + t * 16, 16)] = zi
            slot_buf[pl.ds(k + t * 16, 16)] = zi

        nsb = (k + SB - 1) // SB

        @pl.when(nsb > 0)
        def _():
            pltpu.async_copy(hs_hbm.at[src_buf.at[pl.ds(0, SB)]],
                             hbuf2.at[0], sem)

        def sb_body(g, _):
            par = g & 1
            pltpu.make_async_copy(hs_hbm.at[pl.ds(0, SB)],
                                  hbuf2.at[par], sem).wait()

            @pl.when(g + 1 < nsb)
            def _prefetch():
                off2 = (g + 1) * SB
                pltpu.async_copy(hs_hbm.at[src_buf.at[pl.ds(off2, SB)]],
                                 hbuf2.at[1 - par], sem)

            hb = hbuf2.at[par]

            def q_body(q, _q):
                off = g * SB + q * 16
                qrow = q * 16
                slv = slot_buf[pl.ds(off, 16)]
                valid = (off + iota) < k
                for hh in range(HEADS):
                    ssrc = plsc.load_gather(
                        hb, [iota + qrow, jnp.full((16,), EMB + hh, jnp.int32)])
                    sdst = plsc.load_gather(
                        sdst_buf, [jnp.full((16,), hh, jnp.int32), slv])
                    a = ssrc + sdst
                    a = jnp.where(a >= 0.0, a, 0.2 * a)
                    ex = jnp.where(valid, jnp.exp(a), 0.0)
                    ex_buf[hh] = ex
                for j in range(16):
                    slot_j = slv[j]
                    exj = plsc.load_gather(
                        ex_buf, [iota, jnp.full((16,), j, jnp.int32)])
                    plsc.addupdate(acc.at[slot_j, pl.ds(EMB, 16)], exj)
                    for t in range(16):
                        hv = hb[qrow + j, pl.ds(t * 16, 16)]
                        plsc.addupdate(acc.at[slot_j, pl.ds(t * 16, 16)],
                                       hv * exj[t // 2])
                return _q

            return lax.fori_loop(0, SB // 16, q_body, 0)

        with jax.named_scope("p2_accum"):
            lax.fori_loop(0, nsb, sb_body, 0)

        with jax.named_scope("p3_out"):
            pltpu.sync_copy(acc, out_hbm.at[wid])

    return sc_kernel


def _finish_tc(parts, b2, R, Wfc, bfc2):
    def body(p_ref, b_ref, r_ref, wfc_ref, bfc_ref, o_ref):
        acc = jnp.sum(p_ref[...], axis=0)
        num = acc[:, :EMB]
        den = acc[:, EMB:EMB + HEADS]
        denr = jnp.dot(den, r_ref[...], preferred_element_type=jnp.float32)
        gat = num / (denr + 1e-16) + b_ref[...]
        o_ref[...] = (jnp.dot(gat, wfc_ref[...],
                              preferred_element_type=jnp.float32)
                      + bfc_ref[...])

    return pl.pallas_call(
        body,
        out_shape=jax.ShapeDtypeStruct((B, HIDDEN), jnp.float32),
    )(parts, b2, R, Wfc, bfc2)


_SC_KERNEL = _make_sc_kernel()


def kernel(x, W, a_src, a_dst, b, Wfc, bfc, edge_index, ptr, target_node_idx):
    edges = edge_index.astype(jnp.int32)
    adj = (target_node_idx.astype(jnp.int32) + ptr[:-1].astype(jnp.int32))
    adj64 = jnp.concatenate([adj, jnp.zeros((64 - B,), jnp.int32)])

    eye = jnp.eye(HEADS, dtype=jnp.float32)
    A_src = (a_src[:, :, None] * eye[:, None, :]).reshape(EMB, HEADS)
    A_dst = (a_dst[:, :, None] * eye[:, None, :]).reshape(EMB, HEADS)
    R = jnp.repeat(eye, HEAD_DIM, axis=1)

    hs = _dense_tc(x, W, A_src, A_dst)

    neg1 = jnp.full((N,), -1, jnp.int32)
    zeros_acc = jnp.zeros((B, HSW), jnp.float32)
    parts = _SC_KERNEL(hs, edges, adj64, neg1, zeros_acc)

    out = _finish_tc(parts, b.reshape(1, EMB), R, Wfc, bfc.reshape(1, HIDDEN))
    return out


# speedup vs baseline: 1.0057x; 1.0057x over previous
import functools

import jax
import jax.numpy as jnp
from jax import lax
from jax.experimental import pallas as pl
from jax.experimental.pallas import tpu as pltpu
from jax.experimental.pallas import tpu_sc as plsc

N = 10000
E = 320000
D_IN = 128
HEADS = 8
HEAD_DIM = 32
EMB = 256
HIDDEN = 512
B = 50

NW = 32
ET = E // 128
TPW = ET // NW
EXTRA = ET - TPW * NW
EMAX = (TPW + 1) * 128
NV78 = TPW * 8
HSW = 272
HSP = 384
SB = 48
BUF = EMAX + 128


def _dense_tc(x, W, A_src, A_dst, x_t):
    BLK = 2000

    def body(x_ref, w_ref, as_ref, ad_ref, xt_ref, o_ref, os_ref):
        W_ = w_ref[...]
        WAd = jnp.dot(W_, ad_ref[...], preferred_element_type=jnp.float32)
        Wf = jnp.concatenate(
            [W_,
             jnp.dot(W_, as_ref[...], preferred_element_type=jnp.float32),
             WAd,
             jnp.zeros((D_IN, HSP - HSW), jnp.float32)],
            axis=1)
        o_ref[...] = jnp.dot(x_ref[...], Wf, preferred_element_type=jnp.float32)
        os_ref[...] = jnp.dot(xt_ref[...], WAd,
                              preferred_element_type=jnp.float32)

    return pl.pallas_call(
        body,
        grid=(N // BLK,),
        in_specs=[
            pl.BlockSpec((BLK, D_IN), lambda i: (i, 0)),
            pl.BlockSpec((D_IN, EMB), lambda i: (0, 0)),
            pl.BlockSpec((EMB, HEADS), lambda i: (0, 0)),
            pl.BlockSpec((EMB, HEADS), lambda i: (0, 0)),
            pl.BlockSpec((64, D_IN), lambda i: (0, 0)),
        ],
        out_specs=[pl.BlockSpec((BLK, HSP), lambda i: (i, 0)),
                   pl.BlockSpec((64, HEADS), lambda i: (0, 0))],
        out_shape=[jax.ShapeDtypeStruct((N, HSP), jnp.float32),
                   jax.ShapeDtypeStruct((64, HEADS), jnp.float32)],
    )(x, W, A_src, A_dst, x_t)


def _make_sc_kernel():
    mesh = plsc.VectorSubcoreMesh(core_axis_name="c", subcore_axis_name="s")

    @functools.partial(
        pl.kernel,
        mesh=mesh,
        out_type=jax.ShapeDtypeStruct((NW, B, HSW), jnp.float32),
        compiler_params=pltpu.CompilerParams(needs_layout_passes=False),
        scratch_types=[
            pltpu.VMEM((N,), jnp.int32),
            pltpu.VMEM((64,), jnp.int32),
            pltpu.VMEM((2, EMAX), jnp.int32),
            pltpu.VMEM((BUF,), jnp.int32),
            pltpu.VMEM((BUF,), jnp.int32),
            pltpu.VMEM((2, SB, HSP), jnp.float32),
            pltpu.VMEM((64, HEADS), jnp.float32),
            pltpu.VMEM((16, 16), jnp.float32),
            pltpu.VMEM((B, HSW), jnp.float32),
            pltpu.SemaphoreType.DMA,
            pltpu.SemaphoreType.DMA,
            pltpu.SemaphoreType.DMA,
            pltpu.SemaphoreType.DMA,
            pltpu.SemaphoreType.DMA,
            pltpu.SemaphoreType.DMA,
        ],
    )
    def sc_kernel(hs_hbm, edge_hbm, adj_hbm, sdstt_hbm, neg_hbm, zero_hbm,
                  out_hbm, slot_tab, adj_v, e0, src_buf, slot_buf, hbuf2,
                  sdst_v, ex_buf, acc, sem, semE, sem_slot, sem_acc,
                  sem_adj, sem_sdst):
        cid = lax.axis_index("c")
        sid = lax.axis_index("s")
        wid = sid * 2 + cid
        bt = TPW * wid + jnp.minimum(wid, EXTRA)
        base = bt * 128
        iota = lax.iota(jnp.int32, 16)
        zf = jnp.zeros((16,), jnp.float32)

        h_adj = pltpu.async_copy(adj_hbm, adj_v, sem_adj)
        h_slot = pltpu.async_copy(neg_hbm, slot_tab, sem_slot)
        h_acc = pltpu.async_copy(zero_hbm, acc, sem_acc)
        h_sdst = pltpu.async_copy(sdstt_hbm, sdst_v, sem_sdst)
        for r in range(8, 16):
            ex_buf[r] = zf

        @pl.when(wid < NW - 1)
        def _():
            pltpu.async_copy(edge_hbm.at[:, pl.ds(base, EMAX)], e0, semE)

        @pl.when(wid == NW - 1)
        def _():
            pltpu.async_copy(edge_hbm.at[:, pl.ds(base, TPW * 128)],
                             e0.at[:, pl.ds(0, TPW * 128)], semE)
            pltpu.async_copy(edge_hbm.at[:, pl.ds(0, 128)],
                             e0.at[:, pl.ds(TPW * 128, 128)], semE)

        scope_p0a = jax.named_scope("p0a_adj")
        scope_p0a.__enter__()
        h_adj.wait()
        h_slot.wait()
        for t in range(4):
            av = adj_v[pl.ds(t * 16, 16)]
            sl = iota + (t * 16)
            plsc.store_scatter(slot_tab, [av], sl, mask=sl < B)
        scope_p0a.__exit__(None, None, None)
        with jax.named_scope("p0c_ewait"):
            pltpu.make_async_copy(edge_hbm.at[:, pl.ds(0, EMAX)],
                                  e0, semE).wait()

        def vec_body(vi, kk):
            srcv = e0[0, pl.ds(vi * 16, 16)]
            dstv = e0[1, pl.ds(vi * 16, 16)]
            slv = plsc.load_gather(slot_tab, [dstv])
            m = slv >= 0
            mi = m.astype(jnp.int32)
            incl = jnp.cumsum(mi)
            idxv = kk + (incl - mi)
            plsc.store_scatter(src_buf, [idxv], srcv, mask=m)
            plsc.store_scatter(slot_buf, [idxv], slv, mask=m)
            return kk + incl[15]

        scope_p1 = jax.named_scope("p1_filter")
        scope_p1.__enter__()
        k = lax.fori_loop(0, NV78, vec_body, jnp.int32(0), unroll=16)
        nv = NV78 + 8 * jnp.int32(wid < EXTRA)
        k = lax.fori_loop(NV78, nv, vec_body, k)
        scope_p1.__exit__(None, None, None)
        h_acc.wait()
        h_sdst.wait()

        zi = jnp.zeros((16,), jnp.int32)
        for t in range(SB // 16):
            src_buf[pl.ds(k + t * 16, 16)] = zi
            slot_buf[pl.ds(k + t * 16, 16)] = zi

        nsb = (k + SB - 1) // SB

        @pl.when(nsb > 0)
        def _():
            pltpu.async_copy(hs_hbm.at[src_buf.at[pl.ds(0, SB)]],
                             hbuf2.at[0], sem)

        def sb_body(g, _):
            par = g & 1
            pltpu.make_async_copy(hs_hbm.at[pl.ds(0, SB)],
                                  hbuf2.at[par], sem).wait()

            @pl.when(g + 1 < nsb)
            def _prefetch():
                off2 = (g + 1) * SB
                pltpu.async_copy(hs_hbm.at[src_buf.at[pl.ds(off2, SB)]],
                                 hbuf2.at[1 - par], sem)

            hb = hbuf2.at[par]

            def q_body(q, _q):
                off = g * SB + q * 16
                qrow = q * 16
                slv = slot_buf[pl.ds(off, 16)]
                valid = (off + iota) < k
                for hh in range(HEADS):
                    ssrc = plsc.load_gather(
                        hb, [iota + qrow, jnp.full((16,), EMB + hh, jnp.int32)])
                    sdst = plsc.load_gather(
                        sdst_v, [slv, jnp.full((16,), hh, jnp.int32)])
                    a = ssrc + sdst
                    a = jnp.where(a >= 0.0, a, 0.2 * a)
                    ex = jnp.where(valid, jnp.exp(a), 0.0)
                    ex_buf[hh] = ex
                for j in range(16):
                    slot_j = slv[j]
                    exj = plsc.load_gather(
                        ex_buf, [iota, jnp.full((16,), j, jnp.int32)])
                    plsc.addupdate(acc.at[slot_j, pl.ds(EMB, 16)], exj)
                    for t in range(16):
                        hv = hb[qrow + j, pl.ds(t * 16, 16)]
                        plsc.addupdate(acc.at[slot_j, pl.ds(t * 16, 16)],
                                       hv * exj[t // 2])
                return _q

            return lax.fori_loop(0, SB // 16, q_body, 0)

        with jax.named_scope("p2_accum"):
            lax.fori_loop(0, nsb, sb_body, 0)

        with jax.named_scope("p3_out"):
            pltpu.sync_copy(acc, out_hbm.at[wid])

    return sc_kernel


def _finish_tc(parts, b2, R, Wfc, bfc2):
    def body(p_ref, b_ref, r_ref, wfc_ref, bfc_ref, o_ref):
        acc = jnp.sum(p_ref[...], axis=0)
        num = acc[:, :EMB]
        den = acc[:, EMB:EMB + HEADS]
        denr = jnp.dot(den, r_ref[...], preferred_element_type=jnp.float32)
        gat = num / (denr + 1e-16) + b_ref[...]
        o_ref[...] = (jnp.dot(gat, wfc_ref[...],
                              preferred_element_type=jnp.float32)
                      + bfc_ref[...])

    return pl.pallas_call(
        body,
        out_shape=jax.ShapeDtypeStruct((B, HIDDEN), jnp.float32),
    )(parts, b2, R, Wfc, bfc2)


_SC_KERNEL = _make_sc_kernel()


def kernel(x, W, a_src, a_dst, b, Wfc, bfc, edge_index, ptr, target_node_idx):
    edges = edge_index.astype(jnp.int32)
    adj = (target_node_idx.astype(jnp.int32) + ptr[:-1].astype(jnp.int32))
    adj64 = jnp.concatenate([adj, jnp.zeros((64 - B,), jnp.int32)])

    eye = jnp.eye(HEADS, dtype=jnp.float32)
    A_src = (a_src[:, :, None] * eye[:, None, :]).reshape(EMB, HEADS)
    A_dst = (a_dst[:, :, None] * eye[:, None, :]).reshape(EMB, HEADS)
    R = jnp.repeat(eye, HEAD_DIM, axis=1)

    x_t = x[adj64]
    hs, sdst_t = _dense_tc(x, W, A_src, A_dst, x_t)

    neg1 = jnp.full((N,), -1, jnp.int32)
    zeros_acc = jnp.zeros((B, HSW), jnp.float32)
    parts = _SC_KERNEL(hs, edges, adj64, sdst_t, neg1, zeros_acc)

    out = _finish_tc(parts, b.reshape(1, EMB), R, Wfc, bfc.reshape(1, HIDDEN))
    return out


# speedup vs baseline: 1.0342x; 1.0284x over previous
import functools

import jax
import jax.numpy as jnp
from jax import lax
from jax.experimental import pallas as pl
from jax.experimental.pallas import tpu as pltpu
from jax.experimental.pallas import tpu_sc as plsc

N = 10000
E = 320000
D_IN = 128
HEADS = 8
HEAD_DIM = 32
EMB = 256
HIDDEN = 512
B = 50

NW = 32
ET = E // 128
TPW = ET // NW
EXTRA = ET - TPW * NW
EMAX = (TPW + 1) * 128
NV78 = TPW * 8
HSW = 272
HSP = 256
SB = 48
BUF = EMAX + 128


def _dense_tc(x, W, A_src, A_dst, x_t):
    BLK = 2000

    def body(x_ref, w_ref, as_ref, ad_ref, xt_ref, o_ref, os_ref):
        W_ = w_ref[...]
        WAd = jnp.dot(W_, ad_ref[...], preferred_element_type=jnp.float32)
        Wf = jnp.concatenate(
            [W_, jnp.dot(W_, as_ref[...], preferred_element_type=jnp.float32)],
            axis=1)
        hs = jnp.dot(x_ref[...], Wf, preferred_element_type=jnp.float32)
        hb = hs[:, :EMB].astype(jnp.bfloat16)
        lo = lax.bitcast_convert_type(hb[:, :128], jnp.uint16).astype(jnp.int32)
        hi = lax.bitcast_convert_type(hb[:, 128:], jnp.uint16).astype(jnp.int32)
        packed = lo | (hi << 16)
        sbits = lax.bitcast_convert_type(hs[:, EMB:EMB + HEADS], jnp.int32)
        o_ref[...] = jnp.concatenate(
            [packed, sbits, jnp.zeros((BLK, HSP - 128 - HEADS), jnp.int32)],
            axis=1)
        os_ref[...] = jnp.dot(xt_ref[...], WAd,
                              preferred_element_type=jnp.float32)

    return pl.pallas_call(
        body,
        grid=(N // BLK,),
        in_specs=[
            pl.BlockSpec((BLK, D_IN), lambda i: (i, 0)),
            pl.BlockSpec((D_IN, EMB), lambda i: (0, 0)),
            pl.BlockSpec((EMB, HEADS), lambda i: (0, 0)),
            pl.BlockSpec((EMB, HEADS), lambda i: (0, 0)),
            pl.BlockSpec((64, D_IN), lambda i: (0, 0)),
        ],
        out_specs=[pl.BlockSpec((BLK, HSP), lambda i: (i, 0)),
                   pl.BlockSpec((64, HEADS), lambda i: (0, 0))],
        out_shape=[jax.ShapeDtypeStruct((N, HSP), jnp.int32),
                   jax.ShapeDtypeStruct((64, HEADS), jnp.float32)],
    )(x, W, A_src, A_dst, x_t)


def _make_sc_kernel():
    mesh = plsc.VectorSubcoreMesh(core_axis_name="c", subcore_axis_name="s")

    @functools.partial(
        pl.kernel,
        mesh=mesh,
        out_type=jax.ShapeDtypeStruct((NW, B, HSW), jnp.float32),
        compiler_params=pltpu.CompilerParams(needs_layout_passes=False),
        scratch_types=[
            pltpu.VMEM((N,), jnp.int32),
            pltpu.VMEM((64,), jnp.int32),
            pltpu.VMEM((2, EMAX), jnp.int32),
            pltpu.VMEM((BUF,), jnp.int32),
            pltpu.VMEM((BUF,), jnp.int32),
            pltpu.VMEM((2, SB, HSP), jnp.int32),
            pltpu.VMEM((64, HEADS), jnp.float32),
            pltpu.VMEM((16, 16), jnp.float32),
            pltpu.VMEM((B, HSW), jnp.float32),
            pltpu.SemaphoreType.DMA,
            pltpu.SemaphoreType.DMA,
            pltpu.SemaphoreType.DMA,
            pltpu.SemaphoreType.DMA,
            pltpu.SemaphoreType.DMA,
            pltpu.SemaphoreType.DMA,
        ],
    )
    def sc_kernel(hs_hbm, edge_hbm, adj_hbm, sdstt_hbm, neg_hbm, zero_hbm,
                  out_hbm, slot_tab, adj_v, e0, src_buf, slot_buf, hbuf2,
                  sdst_v, ex_buf, acc, sem, semE, sem_slot, sem_acc,
                  sem_adj, sem_sdst):
        cid = lax.axis_index("c")
        sid = lax.axis_index("s")
        wid = sid * 2 + cid
        bt = TPW * wid + jnp.minimum(wid, EXTRA)
        base = bt * 128
        iota = lax.iota(jnp.int32, 16)
        zf = jnp.zeros((16,), jnp.float32)

        h_adj = pltpu.async_copy(adj_hbm, adj_v, sem_adj)
        h_slot = pltpu.async_copy(neg_hbm, slot_tab, sem_slot)
        h_acc = pltpu.async_copy(zero_hbm, acc, sem_acc)
        h_sdst = pltpu.async_copy(sdstt_hbm, sdst_v, sem_sdst)
        for r in range(8, 16):
            ex_buf[r] = zf

        @pl.when(wid < NW - 1)
        def _():
            pltpu.async_copy(edge_hbm.at[:, pl.ds(base, EMAX)], e0, semE)

        @pl.when(wid == NW - 1)
        def _():
            pltpu.async_copy(edge_hbm.at[:, pl.ds(base, TPW * 128)],
                             e0.at[:, pl.ds(0, TPW * 128)], semE)
            pltpu.async_copy(edge_hbm.at[:, pl.ds(0, 128)],
                             e0.at[:, pl.ds(TPW * 128, 128)], semE)

        scope_p0a = jax.named_scope("p0a_adj")
        scope_p0a.__enter__()
        h_adj.wait()
        h_slot.wait()
        for t in range(4):
            av = adj_v[pl.ds(t * 16, 16)]
            sl = iota + (t * 16)
            plsc.store_scatter(slot_tab, [av], sl, mask=sl < B)
        scope_p0a.__exit__(None, None, None)
        with jax.named_scope("p0c_ewait"):
            pltpu.make_async_copy(edge_hbm.at[:, pl.ds(0, EMAX)],
                                  e0, semE).wait()

        def vec_body(vi, kk):
            srcv = e0[0, pl.ds(vi * 16, 16)]
            dstv = e0[1, pl.ds(vi * 16, 16)]
            slv = plsc.load_gather(slot_tab, [dstv])
            m = slv >= 0
            mi = m.astype(jnp.int32)
            incl = jnp.cumsum(mi)
            idxv = kk + (incl - mi)
            plsc.store_scatter(src_buf, [idxv], srcv, mask=m)
            plsc.store_scatter(slot_buf, [idxv], slv, mask=m)
            return kk + incl[15]

        scope_p1 = jax.named_scope("p1_filter")
        scope_p1.__enter__()
        k = lax.fori_loop(0, NV78, vec_body, jnp.int32(0), unroll=16)
        nv = NV78 + 8 * jnp.int32(wid < EXTRA)
        k = lax.fori_loop(NV78, nv, vec_body, k)
        scope_p1.__exit__(None, None, None)
        h_acc.wait()
        h_sdst.wait()

        zi = jnp.zeros((16,), jnp.int32)
        for t in range(SB // 16):
            src_buf[pl.ds(k + t * 16, 16)] = zi
            slot_buf[pl.ds(k + t * 16, 16)] = zi

        nsb = (k + SB - 1) // SB

        @pl.when(nsb > 0)
        def _():
            pltpu.async_copy(hs_hbm.at[src_buf.at[pl.ds(0, SB)]],
                             hbuf2.at[0], sem)

        def sb_body(g, _):
            par = g & 1
            pltpu.make_async_copy(hs_hbm.at[pl.ds(0, SB)],
                                  hbuf2.at[par], sem).wait()

            @pl.when(g + 1 < nsb)
            def _prefetch():
                off2 = (g + 1) * SB
                pltpu.async_copy(hs_hbm.at[src_buf.at[pl.ds(off2, SB)]],
                                 hbuf2.at[1 - par], sem)

            hb = hbuf2.at[par]

            def q_body(q, _q):
                off = g * SB + q * 16
                qrow = q * 16
                slv = slot_buf[pl.ds(off, 16)]
                valid = (off + iota) < k
                for hh in range(HEADS):
                    sbits = plsc.load_gather(
                        hb, [iota + qrow, jnp.full((16,), 128 + hh, jnp.int32)])
                    ssrc = plsc.bitcast(sbits, jnp.float32)
                    sdst = plsc.load_gather(
                        sdst_v, [slv, jnp.full((16,), hh, jnp.int32)])
                    a = ssrc + sdst
                    a = jnp.where(a >= 0.0, a, 0.2 * a)
                    ex = jnp.where(valid, jnp.exp(a), 0.0)
                    ex_buf[hh] = ex
                for j in range(16):
                    slot_j = slv[j]
                    exj = plsc.load_gather(
                        ex_buf, [iota, jnp.full((16,), j, jnp.int32)])
                    plsc.addupdate(acc.at[slot_j, pl.ds(EMB, 16)], exj)
                    for t in range(8):
                        w = hb[qrow + j, pl.ds(t * 16, 16)]
                        lov = plsc.bitcast(w << 16, jnp.float32)
                        hiv = plsc.bitcast(w & jnp.int32(-65536), jnp.float32)
                        plsc.addupdate(acc.at[slot_j, pl.ds(t * 16, 16)],
                                       lov * exj[t // 2])
                        plsc.addupdate(acc.at[slot_j, pl.ds(128 + t * 16, 16)],
                                       hiv * exj[4 + t // 2])
                return _q

            return lax.fori_loop(0, SB // 16, q_body, 0)

        with jax.named_scope("p2_accum"):
            lax.fori_loop(0, nsb, sb_body, 0)

        with jax.named_scope("p3_out"):
            pltpu.sync_copy(acc, out_hbm.at[wid])

    return sc_kernel


def _finish_tc(parts, b2, R, Wfc, bfc2):
    def body(p_ref, b_ref, r_ref, wfc_ref, bfc_ref, o_ref):
        acc = jnp.sum(p_ref[...], axis=0)
        num = acc[:, :EMB]
        den = acc[:, EMB:EMB + HEADS]
        denr = jnp.dot(den, r_ref[...], preferred_element_type=jnp.float32)
        gat = num / (denr + 1e-16) + b_ref[...]
        o_ref[...] = (jnp.dot(gat, wfc_ref[...],
                              preferred_element_type=jnp.float32)
                      + bfc_ref[...])

    return pl.pallas_call(
        body,
        out_shape=jax.ShapeDtypeStruct((B, HIDDEN), jnp.float32),
    )(parts, b2, R, Wfc, bfc2)


_SC_KERNEL = _make_sc_kernel()


def kernel(x, W, a_src, a_dst, b, Wfc, bfc, edge_index, ptr, target_node_idx):
    edges = edge_index.astype(jnp.int32)
    adj = (target_node_idx.astype(jnp.int32) + ptr[:-1].astype(jnp.int32))
    adj64 = jnp.concatenate([adj, jnp.zeros((64 - B,), jnp.int32)])

    eye = jnp.eye(HEADS, dtype=jnp.float32)
    A_src = (a_src[:, :, None] * eye[:, None, :]).reshape(EMB, HEADS)
    A_dst = (a_dst[:, :, None] * eye[:, None, :]).reshape(EMB, HEADS)
    R = jnp.repeat(eye, HEAD_DIM, axis=1)

    x_t = x[adj64]
    hs, sdst_t = _dense_tc(x, W, A_src, A_dst, x_t)

    neg1 = jnp.full((N,), -1, jnp.int32)
    zeros_acc = jnp.zeros((B, HSW), jnp.float32)
    parts = _SC_KERNEL(hs, edges, adj64, sdst_t, neg1, zeros_acc)

    out = _finish_tc(parts, b.reshape(1, EMB), R, Wfc, bfc.reshape(1, HIDDEN))
    return out


# speedup vs baseline: 1.1802x; 1.1412x over previous
import functools

import jax
import jax.numpy as jnp
from jax import lax
from jax.experimental import pallas as pl
from jax.experimental.pallas import tpu as pltpu
from jax.experimental.pallas import tpu_sc as plsc

N = 10000
E = 320000
D_IN = 128
HEADS = 8
HEAD_DIM = 32
EMB = 256
HIDDEN = 512
B = 50

NW = 32
ET = E // 128
TPW = ET // NW
EXTRA = ET - TPW * NW
EMAX = (TPW + 1) * 128
NV78 = TPW * 8
HSW = 272
HSP = 256
SB = 64
BUF = EMAX + 128


def _dense_tc(x, W, A_src, A_dst, x_t):
    BLK = 2000

    def body(x_ref, w_ref, as_ref, ad_ref, xt_ref, o_ref, os_ref):
        W_ = w_ref[...]
        WAd = jnp.dot(W_, ad_ref[...], preferred_element_type=jnp.float32)
        Wf = jnp.concatenate(
            [W_, jnp.dot(W_, as_ref[...], preferred_element_type=jnp.float32)],
            axis=1)
        hs = jnp.dot(x_ref[...], Wf, preferred_element_type=jnp.float32)
        hb = hs[:, :EMB].astype(jnp.bfloat16)
        lo = lax.bitcast_convert_type(hb[:, :128], jnp.uint16).astype(jnp.int32)
        hi = lax.bitcast_convert_type(hb[:, 128:], jnp.uint16).astype(jnp.int32)
        packed = lo | (hi << 16)
        sbits = lax.bitcast_convert_type(hs[:, EMB:EMB + HEADS], jnp.int32)
        o_ref[...] = jnp.concatenate(
            [packed, sbits, jnp.zeros((BLK, HSP - 128 - HEADS), jnp.int32)],
            axis=1)
        os_ref[...] = jnp.dot(xt_ref[...], WAd,
                              preferred_element_type=jnp.float32)

    return pl.pallas_call(
        body,
        grid=(N // BLK,),
        in_specs=[
            pl.BlockSpec((BLK, D_IN), lambda i: (i, 0)),
            pl.BlockSpec((D_IN, EMB), lambda i: (0, 0)),
            pl.BlockSpec((EMB, HEADS), lambda i: (0, 0)),
            pl.BlockSpec((EMB, HEADS), lambda i: (0, 0)),
            pl.BlockSpec((64, D_IN), lambda i: (0, 0)),
        ],
        out_specs=[pl.BlockSpec((BLK, HSP), lambda i: (i, 0)),
                   pl.BlockSpec((64, HEADS), lambda i: (0, 0))],
        out_shape=[jax.ShapeDtypeStruct((N, HSP), jnp.int32),
                   jax.ShapeDtypeStruct((64, HEADS), jnp.float32)],
    )(x, W, A_src, A_dst, x_t)


def _make_sc_kernel():
    mesh = plsc.VectorSubcoreMesh(core_axis_name="c", subcore_axis_name="s")

    @functools.partial(
        pl.kernel,
        mesh=mesh,
        out_type=jax.ShapeDtypeStruct((NW, B, HSW), jnp.float32),
        compiler_params=pltpu.CompilerParams(needs_layout_passes=False),
        scratch_types=[
            pltpu.VMEM((N,), jnp.int32),
            pltpu.VMEM((64,), jnp.int32),
            pltpu.VMEM((2, EMAX), jnp.int32),
            pltpu.VMEM((BUF,), jnp.int32),
            pltpu.VMEM((BUF,), jnp.int32),
            pltpu.VMEM((2, SB, HSP), jnp.int32),
            pltpu.VMEM((64, HEADS), jnp.float32),
            pltpu.VMEM((16, 16), jnp.float32),
            pltpu.VMEM((B, HSW), jnp.float32),
            pltpu.SemaphoreType.DMA,
            pltpu.SemaphoreType.DMA,
            pltpu.SemaphoreType.DMA,
            pltpu.SemaphoreType.DMA,
            pltpu.SemaphoreType.DMA,
            pltpu.SemaphoreType.DMA,
        ],
    )
    def sc_kernel(hs_hbm, edge_hbm, adj_hbm, sdstt_hbm, neg_hbm, zero_hbm,
                  out_hbm, slot_tab, adj_v, e0, src_buf, slot_buf, hbuf2,
                  sdst_v, ex_buf, acc, sem, semE, sem_slot, sem_acc,
                  sem_adj, sem_sdst):
        cid = lax.axis_index("c")
        sid = lax.axis_index("s")
        wid = sid * 2 + cid
        bt = TPW * wid + jnp.minimum(wid, EXTRA)
        base = bt * 128
        iota = lax.iota(jnp.int32, 16)
        zf = jnp.zeros((16,), jnp.float32)

        h_adj = pltpu.async_copy(adj_hbm, adj_v, sem_adj)
        h_slot = pltpu.async_copy(neg_hbm, slot_tab, sem_slot)
        h_acc = pltpu.async_copy(zero_hbm, acc, sem_acc)
        h_sdst = pltpu.async_copy(sdstt_hbm, sdst_v, sem_sdst)
        for r in range(8, 16):
            ex_buf[r] = zf

        @pl.when(wid < NW - 1)
        def _():
            pltpu.async_copy(edge_hbm.at[:, pl.ds(base, EMAX)], e0, semE)

        @pl.when(wid == NW - 1)
        def _():
            pltpu.async_copy(edge_hbm.at[:, pl.ds(base, TPW * 128)],
                             e0.at[:, pl.ds(0, TPW * 128)], semE)
            pltpu.async_copy(edge_hbm.at[:, pl.ds(0, 128)],
                             e0.at[:, pl.ds(TPW * 128, 128)], semE)

        scope_p0a = jax.named_scope("p0a_adj")
        scope_p0a.__enter__()
        h_adj.wait()
        h_slot.wait()
        for t in range(4):
            av = adj_v[pl.ds(t * 16, 16)]
            sl = iota + (t * 16)
            plsc.store_scatter(slot_tab, [av], sl, mask=sl < B)
        scope_p0a.__exit__(None, None, None)
        with jax.named_scope("p0c_ewait"):
            pltpu.make_async_copy(edge_hbm.at[:, pl.ds(0, EMAX)],
                                  e0, semE).wait()

        def vec_body(vi, kk):
            srcv = e0[0, pl.ds(vi * 16, 16)]
            dstv = e0[1, pl.ds(vi * 16, 16)]
            slv = plsc.load_gather(slot_tab, [dstv])
            m = slv >= 0
            mi = m.astype(jnp.int32)
            incl = jnp.cumsum(mi)
            idxv = kk + (incl - mi)
            plsc.store_scatter(src_buf, [idxv], srcv, mask=m)
            plsc.store_scatter(slot_buf, [idxv], slv, mask=m)
            return kk + incl[15]

        scope_p1 = jax.named_scope("p1_filter")
        scope_p1.__enter__()
        k = lax.fori_loop(0, NV78, vec_body, jnp.int32(0), unroll=16)
        nv = NV78 + 8 * jnp.int32(wid < EXTRA)
        k = lax.fori_loop(NV78, nv, vec_body, k)
        scope_p1.__exit__(None, None, None)
        h_acc.wait()
        h_sdst.wait()

        zi = jnp.zeros((16,), jnp.int32)
        for t in range(SB // 16):
            src_buf[pl.ds(k + t * 16, 16)] = zi
            slot_buf[pl.ds(k + t * 16, 16)] = zi

        nsb = (k + SB - 1) // SB

        @pl.when(nsb > 0)
        def _():
            pltpu.async_copy(hs_hbm.at[src_buf.at[pl.ds(0, SB)]],
                             hbuf2.at[0], sem)

        def sb_body(g, _):
            par = g & 1
            pltpu.make_async_copy(hs_hbm.at[pl.ds(0, SB)],
                                  hbuf2.at[par], sem).wait()

            @pl.when(g + 1 < nsb)
            def _prefetch():
                off2 = (g + 1) * SB
                pltpu.async_copy(hs_hbm.at[src_buf.at[pl.ds(off2, SB)]],
                                 hbuf2.at[1 - par], sem)

            hb = hbuf2.at[par]

            def q_body(q, _q):
                off = g * SB + q * 16
                qrow = q * 16
                slv = slot_buf[pl.ds(off, 16)]
                valid = (off + iota) < k
                for hh in range(HEADS):
                    sbits = plsc.load_gather(
                        hb, [iota + qrow, jnp.full((16,), 128 + hh, jnp.int32)])
                    ssrc = plsc.bitcast(sbits, jnp.float32)
                    sdst = plsc.load_gather(
                        sdst_v, [slv, jnp.full((16,), hh, jnp.int32)])
                    a = ssrc + sdst
                    a = jnp.where(a >= 0.0, a, 0.2 * a)
                    ex = jnp.where(valid, jnp.exp(a), 0.0)
                    ex_buf[hh] = ex
                for j in range(16):
                    slot_j = slv[j]
                    exj = plsc.load_gather(
                        ex_buf, [iota, jnp.full((16,), j, jnp.int32)])
                    plsc.addupdate(acc.at[slot_j, pl.ds(EMB, 16)], exj)
                    for t in range(8):
                        w = hb[qrow + j, pl.ds(t * 16, 16)]
                        lov = plsc.bitcast(w << 16, jnp.float32)
                        hiv = plsc.bitcast(w & jnp.int32(-65536), jnp.float32)
                        plsc.addupdate(acc.at[slot_j, pl.ds(t * 16, 16)],
                                       lov * exj[t // 2])
                        plsc.addupdate(acc.at[slot_j, pl.ds(128 + t * 16, 16)],
                                       hiv * exj[4 + t // 2])
                return _q

            return lax.fori_loop(0, SB // 16, q_body, 0)

        with jax.named_scope("p2_accum"):
            lax.fori_loop(0, nsb, sb_body, 0)

        with jax.named_scope("p3_out"):
            pltpu.sync_copy(acc, out_hbm.at[wid])

    return sc_kernel


def _finish_tc(parts, b2, R, Wfc, bfc2):
    def body(p_ref, b_ref, r_ref, wfc_ref, bfc_ref, o_ref):
        acc = jnp.sum(p_ref[...], axis=0)
        num = acc[:, :EMB]
        den = acc[:, EMB:EMB + HEADS]
        denr = jnp.dot(den, r_ref[...], preferred_element_type=jnp.float32)
        gat = num / (denr + 1e-16) + b_ref[...]
        o_ref[...] = (jnp.dot(gat, wfc_ref[...],
                              preferred_element_type=jnp.float32)
                      + bfc_ref[...])

    return pl.pallas_call(
        body,
        out_shape=jax.ShapeDtypeStruct((B, HIDDEN), jnp.float32),
    )(parts, b2, R, Wfc, bfc2)


_SC_KERNEL = _make_sc_kernel()


def kernel(x, W, a_src, a_dst, b, Wfc, bfc, edge_index, ptr, target_node_idx):
    edges = edge_index.astype(jnp.int32)
    adj = (target_node_idx.astype(jnp.int32) + ptr[:-1].astype(jnp.int32))
    adj64 = jnp.concatenate([adj, jnp.zeros((64 - B,), jnp.int32)])

    eye = jnp.eye(HEADS, dtype=jnp.float32)
    A_src = (a_src[:, :, None] * eye[:, None, :]).reshape(EMB, HEADS)
    A_dst = (a_dst[:, :, None] * eye[:, None, :]).reshape(EMB, HEADS)
    R = jnp.repeat(eye, HEAD_DIM, axis=1)

    x_t = x[adj64]
    hs, sdst_t = _dense_tc(x, W, A_src, A_dst, x_t)

    neg1 = jnp.full((N,), -1, jnp.int32)
    zeros_acc = jnp.zeros((B, HSW), jnp.float32)
    parts = _SC_KERNEL(hs, edges, adj64, sdst_t, neg1, zeros_acc)

    out = _finish_tc(parts, b.reshape(1, EMB), R, Wfc, bfc.reshape(1, HIDDEN))
    return out
